# baseline (speedup 1.0000x reference)
import functools
import jax
import jax.numpy as jnp
from jax import lax
from jax.experimental import pallas as pl
from jax.experimental.pallas import tpu as pltpu
from jax.experimental.pallas import tpu_sc as plsc

HEIGHT = 128
WIDTH = 128
SIZE = 8
NEURONS = 16
NUM_W = WIDTH - (SIZE - 1)
NUM_H = HEIGHT - (SIZE - 1)
NUM_ZONES = NUM_H * NUM_W
PATCH = SIZE * SIZE
RPB = 11

SC_ROWS = 33
TC_ROWS = NUM_H - SC_ROWS
Z0 = TC_ROWS * NUM_W
SC_N = NUM_ZONES - Z0

NW = 32
CPT = (SC_N + NW - 1) // NW
CPT_PAD = ((CPT + 7) // 8) * 8
CB = 32
NCHUNK = (CPT + CB - 1) // CB


def _tc_body(x_ref, w_ref, o_ref):
    i = pl.program_id(0)
    base = i * RPB
    xs = x_ref[pl.ds(base, RPB + SIZE - 1), :]

    for rr in range(RPB):
        segs = []
        for dr in range(SIZE):
            row = xs[rr + dr:rr + dr + 1, :]
            for dc in range(SIZE):
                segs.append(row[:, dc:dc + NUM_W])
        PT = jnp.concatenate(segs, axis=0)
        P = PT.T
        prod = w_ref[rr] * P[:, None, :]
        resp = jnp.sum(prod, axis=2)
        m = jnp.max(resp, axis=1, keepdims=True)
        o_ref[rr] = jnp.where(resp >= m, resp, 0.0)


def _tc_part(x, W4):
    out = pl.pallas_call(
        _tc_body,
        grid=(TC_ROWS // RPB,),
        in_specs=[
            pl.BlockSpec((HEIGHT, WIDTH), lambda i: (0, 0)),
            pl.BlockSpec((RPB, NUM_W, NEURONS, PATCH), lambda i: (i, 0, 0, 0)),
        ],
        out_specs=pl.BlockSpec((RPB, NUM_W, NEURONS), lambda i: (i, 0, 0)),
        out_shape=jax.ShapeDtypeStruct((TC_ROWS, NUM_W, NEURONS), jnp.float32),
    )(x, W4)
    return out.reshape(TC_ROWS * NUM_W, NEURONS)


def _sc_body(x_hbm, wsc_hbm, out_hbm, xbuf, wbuf, pbuf, obuf):
    wid = lax.axis_index("s") * 2 + lax.axis_index("c")
    z0 = wid * CPT
    nz = jnp.minimum(CPT, SC_N - z0)

    pltpu.sync_copy(x_hbm, xbuf)

    lane = lax.iota(jnp.int32, 16)
    lane64 = lane * PATCH
    offs = []
    for v in range(4):
        q = lane + 16 * v
        offs.append(((q >> 3) << 7) + (q & 7))

    w4 = wid * (CPT - NUM_W)
    wrap = (w4 >= NUM_W).astype(jnp.int32)
    r0 = TC_ROWS + wid + wrap
    c0 = w4 - wrap * NUM_W

    rc = (r0, c0)
    for b in range(NCHUNK):
        zb = z0 + b * CB
        zbc = jnp.minimum(zb, SC_N - CB)
        off = zb - zbc
        ncb = jnp.clip(nz - b * CB, 0, CB)

        pltpu.sync_copy(wsc_hbm.at[pl.ds(zbc * NEURONS * PATCH,
                                         CB * NEURONS * PATCH)], wbuf)

        def one_zone(j, rc):
            r, c = rc
            base = (r << 7) + c
            wrow = (j + off) * (NEURONS * PATCH)
            resp = jnp.zeros((16,), jnp.float32)
            for q in range(PATCH):
                wv = plsc.load_gather(wbuf, [wrow + q + lane64])
                poff = ((q >> 3) << 7) + (q & 7)
                pq = plsc.load_gather(
                    xbuf, [base + jnp.full((16,), poff, jnp.int32)])
                resp = resp + wv * pq
            m = jnp.max(resp)
            resp = jnp.where(resp >= m, resp, 0.0)
            obuf[pl.ds((b * CB + j) * NEURONS, NEURONS)] = resp
            nc = c + 1
            w = (nc == NUM_W).astype(jnp.int32)
            return (r + w, nc - w * NUM_W)

        rc = lax.fori_loop(0, ncb, one_zone, rc)

    pltpu.sync_copy(obuf, out_hbm.at[pl.ds(wid * CPT_PAD * NEURONS,
                                           CPT_PAD * NEURONS)])


def _sc_part(x, W):
    mesh = plsc.VectorSubcoreMesh(core_axis_name="c", subcore_axis_name="s")
    k = functools.partial(
        pl.kernel,
        out_type=jax.ShapeDtypeStruct((NW * CPT_PAD * NEURONS,), jnp.float32),
        mesh=mesh,
        compiler_params=pltpu.CompilerParams(needs_layout_passes=False),
        scratch_types=[
            pltpu.VMEM((HEIGHT * WIDTH,), jnp.float32),
            pltpu.VMEM((CB * NEURONS * PATCH,), jnp.float32),
            pltpu.VMEM((PATCH,), jnp.float32),
            pltpu.VMEM((CPT_PAD * NEURONS,), jnp.float32),
        ],
    )(_sc_body)
    wsc = W[Z0:].reshape(SC_N * NEURONS * PATCH)
    flat = k(x.reshape(-1), wsc)
    rows = flat.reshape(NW, CPT_PAD, NEURONS)[:, :CPT, :].reshape(-1, NEURONS)
    return rows[:SC_N]


def kernel(x, W):
    W4 = W.reshape(NUM_H, NUM_W, NEURONS, PATCH)
    out_sc = _sc_part(x, W)
    out_tc = _tc_part(x, W4)
    return jnp.concatenate([out_tc, out_sc], axis=0)


# speedup vs baseline: 1.1872x; 1.1872x over previous
import functools
import jax
import jax.numpy as jnp
from jax import lax
from jax.experimental import pallas as pl
from jax.experimental.pallas import tpu as pltpu
from jax.experimental.pallas import tpu_sc as plsc

HEIGHT = 128
WIDTH = 128
SIZE = 8
NEURONS = 16
NUM_W = WIDTH - (SIZE - 1)
NUM_H = HEIGHT - (SIZE - 1)
NUM_ZONES = NUM_H * NUM_W
PATCH = SIZE * SIZE
RPB = 11

SC_ROWS = 11
TC_ROWS = NUM_H - SC_ROWS
Z0 = TC_ROWS * NUM_W
SC_N = NUM_ZONES - Z0

NW = 32
CPT = (SC_N + NW - 1) // NW
CPT_PAD = ((CPT + 7) // 8) * 8
CB = 32
NCHUNK = (CPT + CB - 1) // CB


def _tc_body(x_ref, w_ref, o_ref):
    i = pl.program_id(0)
    base = i * RPB
    xs = x_ref[pl.ds(base, RPB + SIZE - 1), :]

    for rr in range(RPB):
        segs = []
        for dr in range(SIZE):
            row = xs[rr + dr:rr + dr + 1, :]
            for dc in range(SIZE):
                segs.append(row[:, dc:dc + NUM_W])
        PT = jnp.concatenate(segs, axis=0)
        P = PT.T
        prod = w_ref[rr] * P[:, None, :]
        resp = jnp.sum(prod, axis=2)
        m = jnp.max(resp, axis=1, keepdims=True)
        o_ref[rr] = jnp.where(resp >= m, resp, 0.0)


def _tc_part(x, W4):
    out = pl.pallas_call(
        _tc_body,
        grid=(TC_ROWS // RPB,),
        in_specs=[
            pl.BlockSpec((HEIGHT, WIDTH), lambda i: (0, 0)),
            pl.BlockSpec((RPB, NUM_W, NEURONS, PATCH), lambda i: (i, 0, 0, 0)),
        ],
        out_specs=pl.BlockSpec((RPB, NUM_W, NEURONS), lambda i: (i, 0, 0)),
        out_shape=jax.ShapeDtypeStruct((TC_ROWS, NUM_W, NEURONS), jnp.float32),
    )(x, W4)
    return out.reshape(TC_ROWS * NUM_W, NEURONS)


def _sc_body(x_hbm, wsc_hbm, out_hbm, xbuf, wbuf, pbuf, obuf):
    wid = lax.axis_index("s") * 2 + lax.axis_index("c")
    z0 = wid * CPT
    nz = jnp.minimum(CPT, SC_N - z0)

    pltpu.sync_copy(x_hbm, xbuf)

    lane = lax.iota(jnp.int32, 16)
    lane64 = lane * PATCH
    offs = []
    for v in range(4):
        q = lane + 16 * v
        offs.append(((q >> 3) << 7) + (q & 7))

    w4 = wid * (CPT - NUM_W)
    wrap = (w4 >= NUM_W).astype(jnp.int32)
    r0 = TC_ROWS + wid + wrap
    c0 = w4 - wrap * NUM_W

    rc = (r0, c0)
    for b in range(NCHUNK):
        zb = z0 + b * CB
        zbc = jnp.minimum(zb, SC_N - CB)
        off = zb - zbc
        ncb = jnp.clip(nz - b * CB, 0, CB)

        pltpu.sync_copy(wsc_hbm.at[pl.ds(zbc * NEURONS * PATCH,
                                         CB * NEURONS * PATCH)], wbuf)

        def one_zone(j, rc):
            r, c = rc
            base = (r << 7) + c
            wrow = (j + off) * (NEURONS * PATCH)
            resp = jnp.zeros((16,), jnp.float32)
            for q in range(PATCH):
                wv = plsc.load_gather(wbuf, [wrow + q + lane64])
                poff = ((q >> 3) << 7) + (q & 7)
                pq = plsc.load_gather(
                    xbuf, [base + jnp.full((16,), poff, jnp.int32)])
                resp = resp + wv * pq
            m = jnp.max(resp)
            resp = jnp.where(resp >= m, resp, 0.0)
            obuf[pl.ds((b * CB + j) * NEURONS, NEURONS)] = resp
            nc = c + 1
            w = (nc == NUM_W).astype(jnp.int32)
            return (r + w, nc - w * NUM_W)

        rc = lax.fori_loop(0, ncb, one_zone, rc)

    pltpu.sync_copy(obuf, out_hbm.at[pl.ds(wid * CPT_PAD * NEURONS,
                                           CPT_PAD * NEURONS)])


def _sc_part(x, W):
    mesh = plsc.VectorSubcoreMesh(core_axis_name="c", subcore_axis_name="s")
    k = functools.partial(
        pl.kernel,
        out_type=jax.ShapeDtypeStruct((NW * CPT_PAD * NEURONS,), jnp.float32),
        mesh=mesh,
        compiler_params=pltpu.CompilerParams(needs_layout_passes=False),
        scratch_types=[
            pltpu.VMEM((HEIGHT * WIDTH,), jnp.float32),
            pltpu.VMEM((CB * NEURONS * PATCH,), jnp.float32),
            pltpu.VMEM((PATCH,), jnp.float32),
            pltpu.VMEM((CPT_PAD * NEURONS,), jnp.float32),
        ],
    )(_sc_body)
    wsc = W[Z0:].reshape(SC_N * NEURONS * PATCH)
    flat = k(x.reshape(-1), wsc)
    rows = flat.reshape(NW, CPT_PAD, NEURONS)[:, :CPT, :].reshape(-1, NEURONS)
    return rows[:SC_N]


def kernel(x, W):
    W4 = W.reshape(NUM_H, NUM_W, NEURONS, PATCH)
    out_sc = _sc_part(x, W)
    out_tc = _tc_part(x, W4)
    return jnp.concatenate([out_tc, out_sc], axis=0)
